# Initial kernel scaffold; baseline (speedup 1.0000x reference)
import jax
import jax.numpy as jnp
from jax.experimental import pallas as pl


def kernel(x, router_w, router_b, dw_w, dw_b, pw_w, pw_b):
    raise NotImplementedError("write your pallas kernel here")


# speedup vs baseline: 7.9990x; 7.9990x over previous
import jax
import jax.numpy as jnp
from jax.experimental import pallas as pl

_H = 32
_W = 32
_C = 384
_E = 8
_HW = _H * _W


def _moe_body(xp_ref, rw_ref, rb_ref, dw9_ref, dwb_ref, pwT_ref, pwb_ref,
              out_ref):
    interior = xp_ref[0, 1:_H + 1, 1:_W + 1, :]
    flat = interior.reshape(_HW, _C)
    pooled = jnp.sum(flat, axis=0, keepdims=True) * (1.0 / _HW)
    logits = []
    for e in range(_E):
        le = jnp.sum(rw_ref[e:e + 1, :] * pooled) + rb_ref[0, e]
        logits.append(le)
    m1 = logits[0]
    i1 = jnp.int32(0)
    for e in range(1, _E):
        hit = logits[e] > m1
        i1 = jnp.where(hit, jnp.int32(e), i1)
        m1 = jnp.where(hit, logits[e], m1)
    m2 = jnp.float32(-jnp.inf)
    i2 = jnp.int32(0)
    for e in range(_E):
        hit = (jnp.int32(e) != i1) & (logits[e] > m2)
        i2 = jnp.where(hit, jnp.int32(e), i2)
        m2 = jnp.where(hit, logits[e], m2)
    w1 = 1.0 / (1.0 + jnp.exp((m2 - m1) * 0.5))
    w2 = 1.0 - w1

    def expert(e):
        dwk = dw9_ref[e]
        acc = None
        for di in range(3):
            for dj in range(3):
                tap = xp_ref[0, di:di + _H, dj:dj + _W, :] \
                    * dwk[3 * di + dj, :][None, None, :]
                acc = tap if acc is None else acc + tap
        h = acc.reshape(_HW, _C) + dwb_ref[e]
        h = h * 0.5 * (1.0 + jax.lax.erf(h * (2.0 ** -0.5)))
        o = jnp.dot(h, pwT_ref[e], preferred_element_type=jnp.float32)
        return o + pwb_ref[e]

    out_ref[0] = w1 * expert(i1) + w2 * expert(i2)


def kernel(x, router_w, router_b, dw_w, dw_b, pw_w, pw_b):
    B, C, H, W = x.shape
    E = router_w.shape[0]
    xp = jnp.pad(x.transpose(0, 2, 3, 1),
                 ((0, 0), (1, 1), (1, 1), (0, 0)))
    dw9 = dw_w.reshape(E, C, 9).transpose(0, 2, 1)
    pwT = pw_w.reshape(E, C, C).transpose(0, 2, 1)
    dwb = dw_b.reshape(E, 1, C)
    pwb = pw_b.reshape(E, 1, C)
    rb = router_b.reshape(1, E)

    out = pl.pallas_call(
        _moe_body,
        grid=(B,),
        in_specs=[
            pl.BlockSpec((1, H + 2, W + 2, C), lambda b: (b, 0, 0, 0)),
            pl.BlockSpec((E, C), lambda b: (0, 0)),
            pl.BlockSpec((1, E), lambda b: (0, 0)),
            pl.BlockSpec((E, 9, C), lambda b: (0, 0, 0)),
            pl.BlockSpec((E, 1, C), lambda b: (0, 0, 0)),
            pl.BlockSpec((E, C, C), lambda b: (0, 0, 0)),
            pl.BlockSpec((E, 1, C), lambda b: (0, 0, 0)),
        ],
        out_specs=pl.BlockSpec((1, H * W, C), lambda b: (b, 0, 0)),
        out_shape=jax.ShapeDtypeStruct((B, H * W, C), jnp.float32),
    )(xp, router_w, rb, dw9, dwb, pwT, pwb)
    return out.reshape(B, H, W, C).transpose(0, 3, 1, 2)
